# baseline (speedup 1.0000x reference)
import jax
import jax.numpy as jnp
from jax import lax
from jax.experimental import pallas as pl

_NB = 64
_SP = 64
_CH = 128
_ROWS = _NB * _SP
_FCB = 1024


def _hyper_body(x_ref, ws_ref, bs_ref, out_ref):
    f32 = jnp.float32
    x = x_ref[...]

    ng = _NB * 72
    jj = lax.broadcasted_iota(jnp.int32, (ng + 16, 1), 0)
    xpos = jj % 8
    ygrp = ((jj - 8) // 8) % 9
    notgap = ygrp <= 7
    bmask = {}
    for dx in (-1, 0, 1):
        valid = (xpos + dx >= 0) & (xpos + dx <= 7) & notgap
        bmask[dx] = valid.astype(f32)

    gz = jnp.zeros((_NB, 8, _CH), f32)
    xg = jnp.concatenate([x.reshape(_NB, 64, _CH), gz], axis=1)
    xg = xg.reshape(ng, _CH)

    for layer in range(3):
        zpad = jnp.zeros((16, _CH), f32)
        pad = jnp.concatenate([zpad, xg, zpad], axis=0)
        base = {}
        for dx in (-1, 0, 1):
            base[dx] = pad[8 + dx:8 + dx + ng + 16, :] * bmask[dx]
        acc = jnp.zeros((ng, _CH), f32)
        t = 0
        for dy in (-1, 0, 1):
            for dx in (-1, 0, 1):
                tap = base[dx][8 + dy * 8:8 + dy * 8 + ng, :]
                acc = acc + jnp.dot(tap, ws_ref[layer, t],
                                    preferred_element_type=f32)
                t += 1
        xg = jnp.maximum(acc + bs_ref[layer, 0:1, :], 0.0)

    out_ref[...] = xg.reshape(_NB, 72, _CH)[:, :64, :].reshape(_ROWS, _CH)


def _mem_body(feats_ref, keysr_ref, keyc_ref, addrc_ref, ctab_ref, out_ref):
    f32 = jnp.float32
    kc = keyc_ref[:, 0:1]
    kr = keysr_ref[0:1, :]
    eq = (kc == kr).astype(f32)
    cnt = jnp.sum(eq, axis=1, keepdims=True)
    sums = jnp.dot(eq, feats_ref[...], preferred_element_type=f32)

    ab = addrc_ref[:, 0:1]
    cols = lax.broadcasted_iota(jnp.int32, (1, 128), 1).astype(f32)
    oh = (ab == cols).astype(f32)
    pb = jnp.dot(oh, ctab_ref[...], preferred_element_type=f32)

    out_ref[...] = sums / cnt + pb


def kernel(style_ids, comp_ids, comp_feats, bias, W1, b1, W2, b2, W3, b3):
    f32 = jnp.float32
    offsets = jnp.array([0, 19, 40], dtype=comp_ids.dtype)
    comp_addrs = comp_ids + offsets[None, :]
    flat_addrs = comp_addrs.reshape(-1)
    keys = (style_ids[:, None] * 68 + comp_addrs).reshape(-1)
    keys_f = keys.astype(f32)
    addrs_f = flat_addrs.astype(f32)

    x = comp_feats.reshape(768, 8192)
    keyc = jnp.broadcast_to(keys_f[:, None], (768, 128))
    keysr = jnp.broadcast_to(keys_f[None, :], (8, 768))
    addrc = jnp.broadcast_to(addrs_f[:, None], (768, 128))
    bias_t = bias.reshape(68, 128, 64).transpose(0, 2, 1).reshape(68, 8192)
    bias_p = jnp.zeros((128, 8192), f32).at[:68].set(bias_t)
    bias_rows = bias_p.reshape(128 * _SP, _CH)
    ws = jnp.stack([w.transpose(2, 3, 1, 0).reshape(9, 128, 128)
                    for w in (W1, W2, W3)])
    bs = jnp.stack([jnp.broadcast_to(b[None, :], (8, 128))
                    for b in (b1, b2, b3)])

    ctab = pl.pallas_call(
        _hyper_body,
        grid=(2,),
        in_specs=[
            pl.BlockSpec((_ROWS, _CH), lambda i: (i, 0)),
            pl.BlockSpec((3, 9, 128, 128), lambda i: (0, 0, 0, 0)),
            pl.BlockSpec((3, 8, 128), lambda i: (0, 0, 0)),
        ],
        out_specs=pl.BlockSpec((_ROWS, _CH), lambda i: (i, 0)),
        out_shape=jax.ShapeDtypeStruct((128 * _SP, _CH), f32),
    )(bias_rows, ws, bs)
    ctab_r = (ctab.reshape(128, 64, 128).transpose(0, 2, 1)
              .reshape(128, 8192))

    out = pl.pallas_call(
        _mem_body,
        grid=(8192 // _FCB,),
        in_specs=[
            pl.BlockSpec((768, _FCB), lambda j: (0, j)),
            pl.BlockSpec((8, 768), lambda j: (0, 0)),
            pl.BlockSpec((768, 128), lambda j: (0, 0)),
            pl.BlockSpec((768, 128), lambda j: (0, 0)),
            pl.BlockSpec((128, _FCB), lambda j: (0, j)),
        ],
        out_specs=pl.BlockSpec((768, _FCB), lambda j: (0, j)),
        out_shape=jax.ShapeDtypeStruct((768, 8192), f32),
    )(x, keysr, keyc, addrc, ctab_r)

    return out.reshape(256, 3, 128, 8, 8)


# speedup vs baseline: 6.3844x; 6.3844x over previous
import jax
import jax.numpy as jnp
from jax import lax
from jax.experimental import pallas as pl

_TNB = 72
_SP = 64
_CH = 128
_TROWS = _TNB * _SP
_FCB = 1024


def _hyper_body(x_ref, ws_ref, bs_ref, out_ref):
    f32 = jnp.float32
    x = x_ref[...]

    ng = _TNB * 72
    jj = lax.broadcasted_iota(jnp.int32, (ng + 16, 1), 0)
    xpos = jj % 8
    ygrp = ((jj - 8) // 8) % 9
    notgap = ygrp <= 7
    bmask = {}
    for dx in (-1, 0, 1):
        valid = (xpos + dx >= 0) & (xpos + dx <= 7) & notgap
        bmask[dx] = valid.astype(f32)

    gz = jnp.zeros((_TNB, 8, _CH), f32)
    xg = jnp.concatenate([x.reshape(_TNB, 64, _CH), gz], axis=1)
    xg = xg.reshape(ng, _CH)

    for layer in range(3):
        zpad = jnp.zeros((16, _CH), f32)
        pad = jnp.concatenate([zpad, xg, zpad], axis=0)
        base = {}
        for dx in (-1, 0, 1):
            base[dx] = pad[8 + dx:8 + dx + ng + 16, :] * bmask[dx]
        acc = jnp.zeros((ng, _CH), f32)
        t = 0
        for dy in (-1, 0, 1):
            for dx in (-1, 0, 1):
                tap = base[dx][8 + dy * 8:8 + dy * 8 + ng, :]
                acc = acc + jnp.dot(tap, ws_ref[layer, t],
                                    preferred_element_type=f32)
                t += 1
        xg = jnp.maximum(acc + bs_ref[layer, 0:1, :], 0.0)

    out_ref[...] = xg.reshape(_TNB, 72, _CH)[:, :64, :].reshape(_TROWS, _CH)


def _mem_body(feats_ref, keysr_ref, keyc_ref, addrc_ref, ctab_ref, out_ref):
    f32 = jnp.float32
    kc = keyc_ref[:, 0:1]
    kr = keysr_ref[0:1, :]
    eq = (kc == kr).astype(f32)
    cnt = jnp.sum(eq, axis=1, keepdims=True)
    sums = jnp.dot(eq, feats_ref[...], preferred_element_type=f32)

    ab = addrc_ref[:, 0:1]
    cols = lax.broadcasted_iota(jnp.int32, (1, _TNB), 1).astype(f32)
    oh = (ab == cols).astype(f32)
    pb = jnp.dot(oh, ctab_ref[...], preferred_element_type=f32)

    out_ref[...] = sums / cnt + pb


def kernel(style_ids, comp_ids, comp_feats, bias, W1, b1, W2, b2, W3, b3):
    f32 = jnp.float32
    offsets = jnp.array([0, 19, 40], dtype=comp_ids.dtype)
    comp_addrs = comp_ids + offsets[None, :]
    flat_addrs = comp_addrs.reshape(-1)
    keys = (style_ids[:, None] * 68 + comp_addrs).reshape(-1)
    keys_f = keys.astype(f32)
    addrs_f = flat_addrs.astype(f32)

    x = comp_feats.reshape(768, 128, 64).transpose(0, 2, 1).reshape(768, 8192)
    keyc = jnp.broadcast_to(keys_f[:, None], (768, 128))
    keysr = jnp.broadcast_to(keys_f[None, :], (8, 768))
    addrc = jnp.broadcast_to(addrs_f[:, None], (768, 128))
    bias_t = bias.reshape(68, 128, 64).transpose(0, 2, 1).reshape(68, 8192)
    bias_p = jnp.zeros((_TNB, 8192), f32).at[:68].set(bias_t)
    bias_rows = bias_p.reshape(_TNB * _SP, _CH)
    ws = jnp.stack([w.transpose(2, 3, 1, 0).reshape(9, 128, 128)
                    for w in (W1, W2, W3)])
    bs = jnp.stack([jnp.broadcast_to(b[None, :], (8, 128))
                    for b in (b1, b2, b3)])

    ctab = pl.pallas_call(
        _hyper_body,
        grid=(1,),
        in_specs=[
            pl.BlockSpec((_TROWS, _CH), lambda i: (0, 0)),
            pl.BlockSpec((3, 9, 128, 128), lambda i: (0, 0, 0, 0)),
            pl.BlockSpec((3, 8, 128), lambda i: (0, 0, 0)),
        ],
        out_specs=pl.BlockSpec((_TROWS, _CH), lambda i: (0, 0)),
        out_shape=jax.ShapeDtypeStruct((_TNB * _SP, _CH), f32),
    )(bias_rows, ws, bs)
    ctab_r = ctab.reshape(_TNB, 8192)

    out = pl.pallas_call(
        _mem_body,
        grid=(8192 // _FCB,),
        in_specs=[
            pl.BlockSpec((768, _FCB), lambda j: (0, j)),
            pl.BlockSpec((8, 768), lambda j: (0, 0)),
            pl.BlockSpec((768, 128), lambda j: (0, 0)),
            pl.BlockSpec((768, 128), lambda j: (0, 0)),
            pl.BlockSpec((_TNB, _FCB), lambda j: (0, j)),
        ],
        out_specs=pl.BlockSpec((768, _FCB), lambda j: (0, j)),
        out_shape=jax.ShapeDtypeStruct((768, 8192), f32),
    )(x, keysr, keyc, addrc, ctab_r)

    return (out.reshape(768, 64, 128).transpose(0, 2, 1)
            .reshape(256, 3, 128, 8, 8))
